# Initial kernel scaffold; baseline (speedup 1.0000x reference)
import jax
import jax.numpy as jnp
from jax.experimental import pallas as pl


def kernel(node_features, mesh_edge_features, senders, receivers, params):
    raise NotImplementedError("write your pallas kernel here")


# speedup vs baseline: 2.1368x; 2.1368x over previous
import functools

import jax
import jax.numpy as jnp
from jax import lax
from jax.experimental import pallas as pl
from jax.experimental.pallas import tpu as pltpu
from jax.experimental.pallas import tpu_sc as plsc

LATENT = 128
TW = 5
N = 10000
E = 320000
NF = 11
EF = 4

NPAD = 10240
NC, NS = 2, 16
NW = NC * NS
CH = 128
CPW = 80
RPW = CPW * CH
EP = NW * RPW
IDXROWS = EP // CH

EBLK = 1024
NBLK = 1024

_f32 = jnp.float32


def _ln(h, g, b):
    mu = jnp.mean(h, axis=-1, keepdims=True)
    var = jnp.mean((h - mu) ** 2, axis=-1, keepdims=True)
    return (h - mu) / jnp.sqrt(var + 1e-5) * g + b


def _enc_body(x_ref, w1_ref, b1_ref, w2_ref, b2_ref, g_ref, b_ref, o_ref):
    h = jnp.maximum(x_ref[...] @ w1_ref[...] + b1_ref[...], 0)
    h = jnp.maximum(h @ w2_ref[...] + b2_ref[...], 0)
    o_ref[...] = _ln(h, g_ref[...], b_ref[...])


def _proj_body(nl_ref, w1s_ref, w1r_ref, p_ref, q_ref):
    nl = nl_ref[...]
    p_ref[...] = nl @ w1s_ref[...]
    q_ref[...] = nl @ w1r_ref[...]


def _edge_body(n_el, gs_ref, gr_ref, *rest):
    el_refs = rest[:n_el]
    w1e_ref, b1_ref, w2_ref, b2_ref, g_ref, b_ref, o_ref = rest[n_el:]
    el = el_refs[0][...]
    for ref in el_refs[1:]:
        el = el + ref[...]
    h = jnp.maximum(gs_ref[...] + gr_ref[...] + el @ w1e_ref[...] + b1_ref[...], 0)
    h = jnp.maximum(h @ w2_ref[...] + b2_ref[...], 0)
    y = _ln(h, g_ref[...], b_ref[...])
    row = pl.program_id(0) * EBLK + lax.broadcasted_iota(jnp.int32, (EBLK, 1), 0)
    o_ref[...] = jnp.where(row < E, y, 0.0)


def _node_body(nl_ref, a0_ref, a1_ref, w1n_ref, w1a_ref, b1_ref, w2_ref,
               b2_ref, g_ref, b_ref, o_ref):
    nl = nl_ref[...]
    aggr = a0_ref[...] + a1_ref[...]
    h = jnp.maximum(nl @ w1n_ref[...] + aggr @ w1a_ref[...] + b1_ref[...], 0)
    h = jnp.maximum(h @ w2_ref[...] + b2_ref[...], 0)
    o_ref[...] = _ln(h, g_ref[...], b_ref[...]) + nl


def _dec_body(nl_ref, c1w_ref, c1b_ref, c2w_ref, c2b_ref, o_ref):
    h = nl_ref[...] @ c1w_ref[...] + c1b_ref[...]
    h = h * jax.nn.sigmoid(h)
    d = h @ c2w_ref[...] + c2b_ref[...]
    dt = lax.broadcasted_iota(jnp.int32, (1, TW), 1).astype(_f32) + 1.0
    o_ref[...] = d * dt


def _row_spec(blk, width):
    return pl.BlockSpec((blk, width), lambda i: (i, 0))


def _full_spec(shape):
    nd = len(shape)
    return pl.BlockSpec(shape, lambda i: (0,) * nd)


def _tc_call(body, grid, in_arrays, in_specs, out_shape, out_specs):
    return pl.pallas_call(
        body,
        grid=(grid,),
        in_specs=in_specs,
        out_specs=out_specs,
        out_shape=out_shape,
    )(*in_arrays)


_SC_MESH = plsc.VectorSubcoreMesh(core_axis_name="c", subcore_axis_name="s")


def _gather_body(p_hbm, q_hbm, si_hbm, ri_hbm, gs_hbm, gr_hbm,
                 si_v, ri_v, rows_a, rows_b, sem_a, sem_b):
    wid = lax.axis_index("s") * NC + lax.axis_index("c")
    pltpu.sync_copy(si_hbm.at[pl.ds(wid * CPW, CPW)], si_v)
    pltpu.sync_copy(ri_hbm.at[pl.ds(wid * CPW, CPW)], ri_v)

    @pl.loop(0, CPW)
    def _(j):
        base = wid * RPW + j * CH
        cp_a = pltpu.async_copy(p_hbm.at[si_v.at[j]], rows_a, sem_a)
        cp_b = pltpu.async_copy(q_hbm.at[ri_v.at[j]], rows_b, sem_b)
        cp_a.wait()
        pltpu.sync_copy(rows_a, gs_hbm.at[pl.ds(base, CH)])
        cp_b.wait()
        pltpu.sync_copy(rows_b, gr_hbm.at[pl.ds(base, CH)])


def _scatter_body(ne_hbm, ri_hbm, out_hbm, ri_v, vals_v, zer_v, accum, sem):
    c = lax.axis_index("c")
    s = lax.axis_index("s")
    wid = s * NC + c

    @pl.loop(0, CH)
    def _(i):
        @pl.loop(0, LATENT // 16)
        def _(k):
            zer_v[i, pl.ds(k * 16, 16)] = jnp.zeros((16,), _f32)

    @pl.loop(0, NPAD // NS // CH)
    def _(t):
        pltpu.sync_copy(zer_v, accum.at[pl.ds(s * (NPAD // NS) + t * CH, CH)])

    plsc.subcore_barrier()
    pltpu.sync_copy(ri_hbm.at[pl.ds(wid * CPW, CPW)], ri_v)

    @pl.loop(0, CPW)
    def _(j):
        pltpu.sync_copy(ne_hbm.at[pl.ds(wid * RPW + j * CH, CH)], vals_v)
        pltpu.sync_copy(vals_v, accum.at[ri_v.at[j]], add=True)

    plsc.subcore_barrier()

    @pl.loop(0, NPAD // NS // CH)
    def _(t):
        off = s * (NPAD // NS) + t * CH
        pltpu.sync_copy(accum.at[pl.ds(off, CH)], out_hbm.at[c].at[pl.ds(off, CH)])


_gather_call = pl.kernel(
    _gather_body,
    out_type=[jax.ShapeDtypeStruct((EP, LATENT), _f32),
              jax.ShapeDtypeStruct((EP, LATENT), _f32)],
    mesh=_SC_MESH,
    scratch_types=[
        pltpu.VMEM((CPW, CH), jnp.int32),
        pltpu.VMEM((CPW, CH), jnp.int32),
        pltpu.VMEM((CH, LATENT), _f32),
        pltpu.VMEM((CH, LATENT), _f32),
        pltpu.SemaphoreType.DMA,
        pltpu.SemaphoreType.DMA,
    ],
)

_scatter_call = pl.kernel(
    _scatter_body,
    out_type=jax.ShapeDtypeStruct((NC, NPAD, LATENT), _f32),
    mesh=_SC_MESH,
    scratch_types=[
        pltpu.VMEM((CPW, CH), jnp.int32),
        pltpu.VMEM((CH, LATENT), _f32),
        pltpu.VMEM((CH, LATENT), _f32),
        pltpu.VMEM_SHARED((NPAD, LATENT), _f32),
        pltpu.SemaphoreType.DMA,
    ],
)


def _mlp_weights(p, w1):
    return (w1, p['b1'].reshape(1, -1), p['w2'], p['b2'].reshape(1, -1),
            p['g'].reshape(1, -1), p['b'].reshape(1, -1))


def kernel(node_features, mesh_edge_features, senders, receivers, params):
    p = params
    si = jnp.concatenate(
        [senders.astype(jnp.int32), jnp.zeros((EP - E,), jnp.int32)]
    ).reshape(IDXROWS, CH)
    ri = jnp.concatenate(
        [receivers.astype(jnp.int32), jnp.zeros((EP - E,), jnp.int32)]
    ).reshape(IDXROWS, CH)
    nf = jnp.zeros((NPAD, 16), _f32).at[:N, :NF].set(node_features)
    ef = jnp.zeros((EP, 8), _f32).at[:E, :EF].set(mesh_edge_features)

    ne = p['node_enc']
    w1n_enc = jnp.zeros((16, LATENT), _f32).at[:NF].set(ne['w1'])
    nw = _mlp_weights(ne, w1n_enc)
    nl = _tc_call(
        _enc_body, NPAD // NBLK,
        (nf,) + nw,
        [_row_spec(NBLK, 16)] + [_full_spec(w.shape) for w in nw],
        jax.ShapeDtypeStruct((NPAD, LATENT), _f32),
        _row_spec(NBLK, LATENT),
    )
    ee = p['edge_enc']
    w1e_enc = jnp.zeros((8, LATENT), _f32).at[:EF].set(ee['w1'])
    ew = _mlp_weights(ee, w1e_enc)
    el0 = _tc_call(
        _enc_body, EP // EBLK,
        (ef,) + ew,
        [_row_spec(EBLK, 8)] + [_full_spec(w.shape) for w in ew],
        jax.ShapeDtypeStruct((EP, LATENT), _f32),
        _row_spec(EBLK, LATENT),
    )

    el_terms = [el0]
    for blk in p['blocks']:
        bw = blk['edge']
        P, Q = _tc_call(
            _proj_body, NPAD // NBLK,
            (nl, bw['w1'][:LATENT], bw['w1'][LATENT:2 * LATENT]),
            [_row_spec(NBLK, LATENT), _full_spec((LATENT, LATENT)),
             _full_spec((LATENT, LATENT))],
            [jax.ShapeDtypeStruct((NPAD, LATENT), _f32)] * 2,
            [_row_spec(NBLK, LATENT)] * 2,
        )
        Gs, Gr = _gather_call(P, Q, si, ri)
        eweights = _mlp_weights(bw, bw['w1'][2 * LATENT:])
        new_e = _tc_call(
            functools.partial(_edge_body, len(el_terms)),
            EP // EBLK,
            (Gs, Gr, *el_terms) + eweights,
            [_row_spec(EBLK, LATENT)] * (2 + len(el_terms))
            + [_full_spec(w.shape) for w in eweights],
            jax.ShapeDtypeStruct((EP, LATENT), _f32),
            _row_spec(EBLK, LATENT),
        )
        el_terms.append(new_e)
        partials = _scatter_call(new_e, ri)
        nb = blk['node']
        nweights = _mlp_weights(nb, nb['w1'][LATENT:])
        nl = _tc_call(
            _node_body, NPAD // NBLK,
            (nl, partials[0], partials[1], nb['w1'][:LATENT]) + nweights,
            [_row_spec(NBLK, LATENT)] * 3 + [_full_spec((LATENT, LATENT))]
            + [_full_spec(w.shape) for w in nweights],
            jax.ShapeDtypeStruct((NPAD, LATENT), _f32),
            _row_spec(NBLK, LATENT),
        )

    out = _tc_call(
        _dec_body, N // 1000,
        (nl, p['c1w'], p['c1b'].reshape(1, -1), p['c2w'], p['c2b'].reshape(1, -1)),
        [_row_spec(1000, LATENT), _full_spec((LATENT, 8)), _full_spec((1, 8)),
         _full_spec((8, TW)), _full_spec((1, TW))],
        jax.ShapeDtypeStruct((N, TW), _f32),
        _row_spec(1000, TW),
    )
    return out


# speedup vs baseline: 2.2161x; 1.0371x over previous
import functools

import jax
import jax.numpy as jnp
from jax import lax
from jax.experimental import pallas as pl
from jax.experimental.pallas import tpu as pltpu
from jax.experimental.pallas import tpu_sc as plsc

LATENT = 128
TW = 5
N = 10000
E = 320000
NF = 11
EF = 4

NPAD = 10240
NC, NS = 2, 16
NW = NC * NS
CH = 128
CPW = 80
RPW = CPW * CH
EP = NW * RPW
IDXROWS = EP // CH

EBLK = 1024
NBLK = 1024

_f32 = jnp.float32


def _ln(h, g, b):
    mu = jnp.mean(h, axis=-1, keepdims=True)
    var = jnp.mean((h - mu) ** 2, axis=-1, keepdims=True)
    return (h - mu) / jnp.sqrt(var + 1e-5) * g + b


def _enc_body(x_ref, w1_ref, b1_ref, w2_ref, b2_ref, g_ref, b_ref, o_ref):
    h = jnp.maximum(x_ref[...] @ w1_ref[...] + b1_ref[...], 0)
    h = jnp.maximum(h @ w2_ref[...] + b2_ref[...], 0)
    o_ref[...] = _ln(h, g_ref[...], b_ref[...])


def _proj_body(nl_ref, w1s_ref, w1r_ref, p_ref, q_ref):
    nl = nl_ref[...]
    p_ref[...] = nl @ w1s_ref[...]
    q_ref[...] = nl @ w1r_ref[...]


def _edge_body(n_el, gs_ref, gr_ref, *rest):
    el_refs = rest[:n_el]
    w1e_ref, b1_ref, w2_ref, b2_ref, g_ref, b_ref, o_ref = rest[n_el:]
    el = el_refs[0][...]
    for ref in el_refs[1:]:
        el = el + ref[...]
    h = jnp.maximum(gs_ref[...] + gr_ref[...] + el @ w1e_ref[...] + b1_ref[...], 0)
    h = jnp.maximum(h @ w2_ref[...] + b2_ref[...], 0)
    y = _ln(h, g_ref[...], b_ref[...])
    row = pl.program_id(0) * EBLK + lax.broadcasted_iota(jnp.int32, (EBLK, 1), 0)
    o_ref[...] = jnp.where(row < E, y, 0.0)


def _node_body(nl_ref, a0_ref, a1_ref, w1n_ref, w1a_ref, b1_ref, w2_ref,
               b2_ref, g_ref, b_ref, o_ref):
    nl = nl_ref[...]
    aggr = a0_ref[...] + a1_ref[...]
    h = jnp.maximum(nl @ w1n_ref[...] + aggr @ w1a_ref[...] + b1_ref[...], 0)
    h = jnp.maximum(h @ w2_ref[...] + b2_ref[...], 0)
    o_ref[...] = _ln(h, g_ref[...], b_ref[...]) + nl


def _dec_body(nl_ref, c1w_ref, c1b_ref, c2w_ref, c2b_ref, o_ref):
    h = nl_ref[...] @ c1w_ref[...] + c1b_ref[...]
    h = h * jax.nn.sigmoid(h)
    d = h @ c2w_ref[...] + c2b_ref[...]
    dt = lax.broadcasted_iota(jnp.int32, (1, TW), 1).astype(_f32) + 1.0
    o_ref[...] = d * dt


def _row_spec(blk, width):
    return pl.BlockSpec((blk, width), lambda i: (i, 0))


def _full_spec(shape):
    nd = len(shape)
    return pl.BlockSpec(shape, lambda i: (0,) * nd)


def _tc_call(body, grid, in_arrays, in_specs, out_shape, out_specs):
    return pl.pallas_call(
        body,
        grid=(grid,),
        in_specs=in_specs,
        out_specs=out_specs,
        out_shape=out_shape,
    )(*in_arrays)


_SC_MESH = plsc.VectorSubcoreMesh(core_axis_name="c", subcore_axis_name="s")


def _gather_body(p_hbm, q_hbm, si_hbm, ri_hbm, gs_hbm, gr_hbm,
                 si_v, ri_v, pbufs, qbufs, gp, gq, wp, wq):
    wid = lax.axis_index("s") * NC + lax.axis_index("c")
    pltpu.sync_copy(si_hbm.at[pl.ds(wid * CPW, CPW)], si_v)
    pltpu.sync_copy(ri_hbm.at[pl.ds(wid * CPW, CPW)], ri_v)

    def issue_gather(j, b):
        pltpu.async_copy(p_hbm.at[si_v.at[j]], pbufs[b], gp[b])
        pltpu.async_copy(q_hbm.at[ri_v.at[j]], qbufs[b], gq[b])

    def wait_gather(b):
        pltpu.make_async_copy(p_hbm.at[si_v.at[0]], pbufs[b], gp[b]).wait()
        pltpu.make_async_copy(q_hbm.at[ri_v.at[0]], qbufs[b], gq[b]).wait()

    def wait_write(b):
        pltpu.make_async_copy(pbufs[b], gs_hbm.at[pl.ds(0, CH)], wp[b]).wait()
        pltpu.make_async_copy(qbufs[b], gr_hbm.at[pl.ds(0, CH)], wq[b]).wait()

    issue_gather(0, 0)

    @pl.loop(0, CPW // 2)
    def _(jj):
        for b in range(2):
            j = jj * 2 + b
            ob = 1 - b
            if b == 0:
                @pl.when(jj >= 1)
                def _():
                    wait_write(ob)
            else:
                wait_write(ob)
            wait_gather(b)
            if b == 0:
                issue_gather(j + 1, ob)
            else:
                @pl.when(j + 1 < CPW)
                def _():
                    issue_gather(j + 1, ob)
            base = wid * RPW + j * CH
            pltpu.async_copy(pbufs[b], gs_hbm.at[pl.ds(base, CH)], wp[b])
            pltpu.async_copy(qbufs[b], gr_hbm.at[pl.ds(base, CH)], wq[b])

    wait_write(1)


def _scatter_body(ne_hbm, ri_hbm, out_hbm, ri_v, vals_v, zer_v, accum, sem):
    c = lax.axis_index("c")
    s = lax.axis_index("s")
    wid = s * NC + c

    @pl.loop(0, CH)
    def _(i):
        @pl.loop(0, LATENT // 16)
        def _(k):
            zer_v[i, pl.ds(k * 16, 16)] = jnp.zeros((16,), _f32)

    @pl.loop(0, NPAD // NS // CH)
    def _(t):
        pltpu.sync_copy(zer_v, accum.at[pl.ds(s * (NPAD // NS) + t * CH, CH)])

    plsc.subcore_barrier()
    pltpu.sync_copy(ri_hbm.at[pl.ds(wid * CPW, CPW)], ri_v)

    @pl.loop(0, CPW)
    def _(j):
        pltpu.sync_copy(ne_hbm.at[pl.ds(wid * RPW + j * CH, CH)], vals_v)
        pltpu.sync_copy(vals_v, accum.at[ri_v.at[j]], add=True)

    plsc.subcore_barrier()

    @pl.loop(0, NPAD // NS // CH)
    def _(t):
        off = s * (NPAD // NS) + t * CH
        pltpu.sync_copy(accum.at[pl.ds(off, CH)], out_hbm.at[c].at[pl.ds(off, CH)])


_gather_call = pl.kernel(
    _gather_body,
    out_type=[jax.ShapeDtypeStruct((EP, LATENT), _f32),
              jax.ShapeDtypeStruct((EP, LATENT), _f32)],
    mesh=_SC_MESH,
    scratch_types=[
        pltpu.VMEM((CPW, CH), jnp.int32),
        pltpu.VMEM((CPW, CH), jnp.int32),
        [pltpu.VMEM((CH, LATENT), _f32)] * 2,
        [pltpu.VMEM((CH, LATENT), _f32)] * 2,
        [pltpu.SemaphoreType.DMA] * 2,
        [pltpu.SemaphoreType.DMA] * 2,
        [pltpu.SemaphoreType.DMA] * 2,
        [pltpu.SemaphoreType.DMA] * 2,
    ],
)

_scatter_call = pl.kernel(
    _scatter_body,
    out_type=jax.ShapeDtypeStruct((NC, NPAD, LATENT), _f32),
    mesh=_SC_MESH,
    scratch_types=[
        pltpu.VMEM((CPW, CH), jnp.int32),
        pltpu.VMEM((CH, LATENT), _f32),
        pltpu.VMEM((CH, LATENT), _f32),
        pltpu.VMEM_SHARED((NPAD, LATENT), _f32),
        pltpu.SemaphoreType.DMA,
    ],
)


def _mlp_weights(p, w1):
    return (w1, p['b1'].reshape(1, -1), p['w2'], p['b2'].reshape(1, -1),
            p['g'].reshape(1, -1), p['b'].reshape(1, -1))


def kernel(node_features, mesh_edge_features, senders, receivers, params):
    p = params
    si = jnp.concatenate(
        [senders.astype(jnp.int32), jnp.zeros((EP - E,), jnp.int32)]
    ).reshape(IDXROWS, CH)
    ri = jnp.concatenate(
        [receivers.astype(jnp.int32), jnp.zeros((EP - E,), jnp.int32)]
    ).reshape(IDXROWS, CH)
    nf = jnp.zeros((NPAD, 16), _f32).at[:N, :NF].set(node_features)
    ef = jnp.zeros((EP, 8), _f32).at[:E, :EF].set(mesh_edge_features)

    ne = p['node_enc']
    w1n_enc = jnp.zeros((16, LATENT), _f32).at[:NF].set(ne['w1'])
    nw = _mlp_weights(ne, w1n_enc)
    nl = _tc_call(
        _enc_body, NPAD // NBLK,
        (nf,) + nw,
        [_row_spec(NBLK, 16)] + [_full_spec(w.shape) for w in nw],
        jax.ShapeDtypeStruct((NPAD, LATENT), _f32),
        _row_spec(NBLK, LATENT),
    )
    ee = p['edge_enc']
    w1e_enc = jnp.zeros((8, LATENT), _f32).at[:EF].set(ee['w1'])
    ew = _mlp_weights(ee, w1e_enc)
    el0 = _tc_call(
        _enc_body, EP // EBLK,
        (ef,) + ew,
        [_row_spec(EBLK, 8)] + [_full_spec(w.shape) for w in ew],
        jax.ShapeDtypeStruct((EP, LATENT), _f32),
        _row_spec(EBLK, LATENT),
    )

    el_terms = [el0]
    for blk in p['blocks']:
        bw = blk['edge']
        P, Q = _tc_call(
            _proj_body, NPAD // NBLK,
            (nl, bw['w1'][:LATENT], bw['w1'][LATENT:2 * LATENT]),
            [_row_spec(NBLK, LATENT), _full_spec((LATENT, LATENT)),
             _full_spec((LATENT, LATENT))],
            [jax.ShapeDtypeStruct((NPAD, LATENT), _f32)] * 2,
            [_row_spec(NBLK, LATENT)] * 2,
        )
        Gs, Gr = _gather_call(P, Q, si, ri)
        eweights = _mlp_weights(bw, bw['w1'][2 * LATENT:])
        new_e = _tc_call(
            functools.partial(_edge_body, len(el_terms)),
            EP // EBLK,
            (Gs, Gr, *el_terms) + eweights,
            [_row_spec(EBLK, LATENT)] * (2 + len(el_terms))
            + [_full_spec(w.shape) for w in eweights],
            jax.ShapeDtypeStruct((EP, LATENT), _f32),
            _row_spec(EBLK, LATENT),
        )
        el_terms.append(new_e)
        partials = _scatter_call(new_e, ri)
        nb = blk['node']
        nweights = _mlp_weights(nb, nb['w1'][LATENT:])
        nl = _tc_call(
            _node_body, NPAD // NBLK,
            (nl, partials[0], partials[1], nb['w1'][:LATENT]) + nweights,
            [_row_spec(NBLK, LATENT)] * 3 + [_full_spec((LATENT, LATENT))]
            + [_full_spec(w.shape) for w in nweights],
            jax.ShapeDtypeStruct((NPAD, LATENT), _f32),
            _row_spec(NBLK, LATENT),
        )

    out = _tc_call(
        _dec_body, N // 1000,
        (nl, p['c1w'], p['c1b'].reshape(1, -1), p['c2w'], p['c2b'].reshape(1, -1)),
        [_row_spec(1000, LATENT), _full_spec((LATENT, 8)), _full_spec((1, 8)),
         _full_spec((8, TW)), _full_spec((1, TW))],
        jax.ShapeDtypeStruct((N, TW), _f32),
        _row_spec(1000, TW),
    )
    return out


# speedup vs baseline: 3.3445x; 1.5092x over previous
import functools

import jax
import jax.numpy as jnp
from jax import lax
from jax.experimental import pallas as pl
from jax.experimental.pallas import tpu as pltpu
from jax.experimental.pallas import tpu_sc as plsc

LATENT = 128
TW = 5
N = 10000
E = 320000
NF = 11
EF = 4

NPAD = 10240
NC, NS = 2, 16
NW = NC * NS
CH = 128
CPW = 80
RPW = CPW * CH
EP = NW * RPW
IDXROWS = EP // CH

EBLK = 1024
NBLK = 1024

_f32 = jnp.float32


def _ln(h, g, b):
    mu = jnp.mean(h, axis=-1, keepdims=True)
    var = jnp.mean((h - mu) ** 2, axis=-1, keepdims=True)
    return (h - mu) / jnp.sqrt(var + 1e-5) * g + b


def _enc_body(x_ref, w1_ref, b1_ref, w2_ref, b2_ref, g_ref, b_ref, o_ref):
    h = jnp.maximum(x_ref[...] @ w1_ref[...] + b1_ref[...], 0)
    h = jnp.maximum(h @ w2_ref[...] + b2_ref[...], 0)
    o_ref[...] = _ln(h, g_ref[...], b_ref[...])


def _proj_body(nl_ref, w1s_ref, w1r_ref, p_ref, q_ref):
    nl = nl_ref[...]
    p_ref[...] = nl @ w1s_ref[...]
    q_ref[...] = nl @ w1r_ref[...]


def _edge_body(n_el, gs_ref, gr_ref, *rest):
    el_refs = rest[:n_el]
    w1e_ref, b1_ref, w2_ref, b2_ref, g_ref, b_ref, o_ref = rest[n_el:]
    el = el_refs[0][...]
    for ref in el_refs[1:]:
        el = el + ref[...]
    h = jnp.maximum(gs_ref[...] + gr_ref[...] + el @ w1e_ref[...] + b1_ref[...], 0)
    h = jnp.maximum(h @ w2_ref[...] + b2_ref[...], 0)
    y = _ln(h, g_ref[...], b_ref[...])
    row = pl.program_id(0) * EBLK + lax.broadcasted_iota(jnp.int32, (EBLK, 1), 0)
    o_ref[...] = jnp.where(row < E, y, 0.0)


def _node_body(nl_ref, a0_ref, a1_ref, w1n_ref, w1a_ref, b1_ref, w2_ref,
               b2_ref, g_ref, b_ref, o_ref):
    nl = nl_ref[...]
    aggr = a0_ref[...] + a1_ref[...]
    h = jnp.maximum(nl @ w1n_ref[...] + aggr @ w1a_ref[...] + b1_ref[...], 0)
    h = jnp.maximum(h @ w2_ref[...] + b2_ref[...], 0)
    o_ref[...] = _ln(h, g_ref[...], b_ref[...]) + nl


def _dec_body(nl_ref, c1w_ref, c1b_ref, c2w_ref, c2b_ref, o_ref):
    h = nl_ref[...] @ c1w_ref[...] + c1b_ref[...]
    h = h * jax.nn.sigmoid(h)
    d = h @ c2w_ref[...] + c2b_ref[...]
    dt = lax.broadcasted_iota(jnp.int32, (1, TW), 1).astype(_f32) + 1.0
    o_ref[...] = d * dt


def _row_spec(blk, width):
    return pl.BlockSpec((blk, width), lambda i: (i, 0))


def _full_spec(shape):
    nd = len(shape)
    return pl.BlockSpec(shape, lambda i: (0,) * nd)


def _tc_call(body, grid, in_arrays, in_specs, out_shape, out_specs):
    return pl.pallas_call(
        body,
        grid=(grid,),
        in_specs=in_specs,
        out_specs=out_specs,
        out_shape=out_shape,
    )(*in_arrays)


_SC_MESH = plsc.VectorSubcoreMesh(core_axis_name="c", subcore_axis_name="s")


CPC = IDXROWS // NS
STRIDE = NPAD // NS


PHCH = CPC // 2


def _gather_one(tab, idx_v, out_hbm, chunk0, bufs, gsem, wsem):
    def issue_gather(j, b):
        pltpu.async_copy(tab.at[idx_v.at[j]], bufs[b], gsem[b])

    def wait_gather(b):
        pltpu.make_async_copy(tab.at[idx_v.at[0]], bufs[b], gsem[b]).wait()

    def wait_write(b):
        pltpu.make_async_copy(bufs[b], out_hbm.at[pl.ds(0, CH)], wsem[b]).wait()

    issue_gather(0, 0)

    @pl.loop(0, PHCH // 2)
    def _(jj):
        for b in range(2):
            j = jj * 2 + b
            ob = 1 - b
            if b == 0:
                @pl.when(jj >= 1)
                def _():
                    wait_write(ob)
            else:
                wait_write(ob)
            wait_gather(b)
            if b == 0:
                issue_gather(j + 1, ob)
            else:
                @pl.when(j + 1 < PHCH)
                def _():
                    issue_gather(j + 1, ob)
            base = (chunk0 + j) * CH
            pltpu.async_copy(bufs[b], out_hbm.at[pl.ds(base, CH)], wsem[b])

    wait_write(1)


def _gather_core(tab, idx_hbm, idx_v, out_hbm, sid, bufs, gsem, wsem):
    for ph in range(2):
        pltpu.sync_copy(idx_hbm.at[pl.ds(sid * CPC + ph * PHCH, PHCH)], idx_v)
        _gather_one(tab, idx_v, out_hbm, sid * CPC + ph * PHCH, bufs, gsem, wsem)


def _gather_body(p_hbm, q_hbm, si_hbm, ri_hbm, gs_hbm, gr_hbm,
                 idx_v, bufs, tab, gsem, wsem):
    c = lax.axis_index("c")
    s = lax.axis_index("s")

    @pl.when(c == 0)
    def _():
        pltpu.sync_copy(p_hbm.at[pl.ds(s * STRIDE, STRIDE)],
                        tab.at[pl.ds(s * STRIDE, STRIDE)])

    @pl.when(c == 1)
    def _():
        pltpu.sync_copy(q_hbm.at[pl.ds(s * STRIDE, STRIDE)],
                        tab.at[pl.ds(s * STRIDE, STRIDE)])

    plsc.subcore_barrier()

    @pl.when(c == 0)
    def _():
        _gather_core(tab, si_hbm, idx_v, gs_hbm, s, bufs, gsem, wsem)

    @pl.when(c == 1)
    def _():
        _gather_core(tab, ri_hbm, idx_v, gr_hbm, s, bufs, gsem, wsem)


def _scatter_body(ne_hbm, ri_hbm, out_hbm, ri_v, vals_v, zer_v, accum, sem):
    c = lax.axis_index("c")
    s = lax.axis_index("s")
    wid = s * NC + c

    @pl.loop(0, CH)
    def _(i):
        @pl.loop(0, LATENT // 16)
        def _(k):
            zer_v[i, pl.ds(k * 16, 16)] = jnp.zeros((16,), _f32)

    @pl.loop(0, NPAD // NS // CH)
    def _(t):
        pltpu.sync_copy(zer_v, accum.at[pl.ds(s * (NPAD // NS) + t * CH, CH)])

    plsc.subcore_barrier()
    pltpu.sync_copy(ri_hbm.at[pl.ds(wid * CPW, CPW)], ri_v)

    @pl.loop(0, CPW)
    def _(j):
        pltpu.sync_copy(ne_hbm.at[pl.ds(wid * RPW + j * CH, CH)], vals_v)
        pltpu.sync_copy(vals_v, accum.at[ri_v.at[j]], add=True)

    plsc.subcore_barrier()

    @pl.loop(0, NPAD // NS // CH)
    def _(t):
        off = s * (NPAD // NS) + t * CH
        pltpu.sync_copy(accum.at[pl.ds(off, CH)], out_hbm.at[c].at[pl.ds(off, CH)])


_gather_call = pl.kernel(
    _gather_body,
    out_type=[jax.ShapeDtypeStruct((EP, LATENT), _f32),
              jax.ShapeDtypeStruct((EP, LATENT), _f32)],
    mesh=_SC_MESH,
    scratch_types=[
        pltpu.VMEM((PHCH, CH), jnp.int32),
        [pltpu.VMEM((CH, LATENT), _f32)] * 2,
        pltpu.VMEM_SHARED((NPAD, LATENT), _f32),
        [pltpu.SemaphoreType.DMA] * 2,
        [pltpu.SemaphoreType.DMA] * 2,
    ],
)

_scatter_call = pl.kernel(
    _scatter_body,
    out_type=jax.ShapeDtypeStruct((NC, NPAD, LATENT), _f32),
    mesh=_SC_MESH,
    scratch_types=[
        pltpu.VMEM((CPW, CH), jnp.int32),
        pltpu.VMEM((CH, LATENT), _f32),
        pltpu.VMEM((CH, LATENT), _f32),
        pltpu.VMEM_SHARED((NPAD, LATENT), _f32),
        pltpu.SemaphoreType.DMA,
    ],
)


def _mlp_weights(p, w1):
    return (w1, p['b1'].reshape(1, -1), p['w2'], p['b2'].reshape(1, -1),
            p['g'].reshape(1, -1), p['b'].reshape(1, -1))


def kernel(node_features, mesh_edge_features, senders, receivers, params):
    p = params
    si = jnp.concatenate(
        [senders.astype(jnp.int32), jnp.zeros((EP - E,), jnp.int32)]
    ).reshape(IDXROWS, CH)
    ri = jnp.concatenate(
        [receivers.astype(jnp.int32), jnp.zeros((EP - E,), jnp.int32)]
    ).reshape(IDXROWS, CH)
    nf = jnp.zeros((NPAD, 16), _f32).at[:N, :NF].set(node_features)

    ne = p['node_enc']
    w1n_enc = jnp.zeros((16, LATENT), _f32).at[:NF].set(ne['w1'])
    nw = _mlp_weights(ne, w1n_enc)
    nl = _tc_call(
        _enc_body, NPAD // NBLK,
        (nf,) + nw,
        [_row_spec(NBLK, 16)] + [_full_spec(w.shape) for w in nw],
        jax.ShapeDtypeStruct((NPAD, LATENT), _f32),
        _row_spec(NBLK, LATENT),
    )
    ee = p['edge_enc']
    ew = _mlp_weights(ee, ee['w1'])
    el0 = _tc_call(
        _enc_body, E // 1000,
        (mesh_edge_features,) + ew,
        [_row_spec(1000, EF)] + [_full_spec(w.shape) for w in ew],
        jax.ShapeDtypeStruct((EP, LATENT), _f32),
        _row_spec(1000, LATENT),
    )

    el_terms = [el0]
    for blk in p['blocks']:
        bw = blk['edge']
        P, Q = _tc_call(
            _proj_body, NPAD // NBLK,
            (nl, bw['w1'][:LATENT], bw['w1'][LATENT:2 * LATENT]),
            [_row_spec(NBLK, LATENT), _full_spec((LATENT, LATENT)),
             _full_spec((LATENT, LATENT))],
            [jax.ShapeDtypeStruct((NPAD, LATENT), _f32)] * 2,
            [_row_spec(NBLK, LATENT)] * 2,
        )
        Gs, Gr = _gather_call(P, Q, si, ri)
        eweights = _mlp_weights(bw, bw['w1'][2 * LATENT:])
        new_e = _tc_call(
            functools.partial(_edge_body, len(el_terms)),
            EP // EBLK,
            (Gs, Gr, *el_terms) + eweights,
            [_row_spec(EBLK, LATENT)] * (2 + len(el_terms))
            + [_full_spec(w.shape) for w in eweights],
            jax.ShapeDtypeStruct((EP, LATENT), _f32),
            _row_spec(EBLK, LATENT),
        )
        el_terms.append(new_e)
        partials = _scatter_call(new_e, ri)
        nb = blk['node']
        nweights = _mlp_weights(nb, nb['w1'][LATENT:])
        nl = _tc_call(
            _node_body, NPAD // NBLK,
            (nl, partials[0], partials[1], nb['w1'][:LATENT]) + nweights,
            [_row_spec(NBLK, LATENT)] * 3 + [_full_spec((LATENT, LATENT))]
            + [_full_spec(w.shape) for w in nweights],
            jax.ShapeDtypeStruct((NPAD, LATENT), _f32),
            _row_spec(NBLK, LATENT),
        )

    out = _tc_call(
        _dec_body, N // 1000,
        (nl, p['c1w'], p['c1b'].reshape(1, -1), p['c2w'], p['c2b'].reshape(1, -1)),
        [_row_spec(1000, LATENT), _full_spec((LATENT, 8)), _full_spec((1, 8)),
         _full_spec((8, TW)), _full_spec((1, TW))],
        jax.ShapeDtypeStruct((N, TW), _f32),
        _row_spec(1000, TW),
    )
    return out


# speedup vs baseline: 3.4204x; 1.0227x over previous
import functools

import jax
import jax.numpy as jnp
from jax import lax
from jax.experimental import pallas as pl
from jax.experimental.pallas import tpu as pltpu
from jax.experimental.pallas import tpu_sc as plsc

LATENT = 128
TW = 5
N = 10000
E = 320000
NF = 11
EF = 4

NPAD = 10240
NC, NS = 2, 16
NW = NC * NS
CH = 128
CPW = 80
RPW = CPW * CH
EP = NW * RPW
IDXROWS = EP // CH

EBLK = 1024
NBLK = 1024

_f32 = jnp.float32


def _ln(h, g, b):
    mu = jnp.mean(h, axis=-1, keepdims=True)
    var = jnp.mean((h - mu) ** 2, axis=-1, keepdims=True)
    return (h - mu) / jnp.sqrt(var + 1e-5) * g + b


def _enc_body(x_ref, w1_ref, b1_ref, w2_ref, b2_ref, g_ref, b_ref, o_ref):
    h = jnp.maximum(x_ref[...] @ w1_ref[...] + b1_ref[...], 0)
    h = jnp.maximum(h @ w2_ref[...] + b2_ref[...], 0)
    o_ref[...] = _ln(h, g_ref[...], b_ref[...]).astype(o_ref.dtype)


def _proj_body(nl_ref, w1s_ref, w1r_ref, p_ref, q_ref):
    nl = nl_ref[...]
    p_ref[...] = nl @ w1s_ref[...]
    q_ref[...] = nl @ w1r_ref[...]


def _edge_body(n_el, gs_ref, gr_ref, *rest):
    el_refs = rest[:n_el]
    w1e_ref, b1_ref, w2_ref, b2_ref, g_ref, b_ref, o_ref = rest[n_el:]
    el = el_refs[0][...].astype(_f32)
    for ref in el_refs[1:]:
        el = el + ref[...]
    h = jnp.maximum(gs_ref[...] + gr_ref[...] + el @ w1e_ref[...] + b1_ref[...], 0)
    h = jnp.maximum(h @ w2_ref[...] + b2_ref[...], 0)
    y = _ln(h, g_ref[...], b_ref[...])
    row = pl.program_id(0) * EBLK + lax.broadcasted_iota(jnp.int32, (EBLK, 1), 0)
    o_ref[...] = jnp.where(row < E, y, 0.0)


def _node_body(nl_ref, a0_ref, a1_ref, w1n_ref, w1a_ref, b1_ref, w2_ref,
               b2_ref, g_ref, b_ref, o_ref):
    nl = nl_ref[...]
    aggr = a0_ref[...] + a1_ref[...]
    h = jnp.maximum(nl @ w1n_ref[...] + aggr @ w1a_ref[...] + b1_ref[...], 0)
    h = jnp.maximum(h @ w2_ref[...] + b2_ref[...], 0)
    o_ref[...] = _ln(h, g_ref[...], b_ref[...]) + nl


def _dec_body(nl_ref, c1w_ref, c1b_ref, c2w_ref, c2b_ref, o_ref):
    h = nl_ref[...] @ c1w_ref[...] + c1b_ref[...]
    h = h * jax.nn.sigmoid(h)
    d = h @ c2w_ref[...] + c2b_ref[...]
    dt = lax.broadcasted_iota(jnp.int32, (1, TW), 1).astype(_f32) + 1.0
    o_ref[...] = d * dt


def _row_spec(blk, width):
    return pl.BlockSpec((blk, width), lambda i: (i, 0))


def _full_spec(shape):
    nd = len(shape)
    return pl.BlockSpec(shape, lambda i: (0,) * nd)


def _tc_call(body, grid, in_arrays, in_specs, out_shape, out_specs):
    return pl.pallas_call(
        body,
        grid=(grid,),
        in_specs=in_specs,
        out_specs=out_specs,
        out_shape=out_shape,
    )(*in_arrays)


_SC_MESH = plsc.VectorSubcoreMesh(core_axis_name="c", subcore_axis_name="s")


CPC = IDXROWS // NS
STRIDE = NPAD // NS


PHCH = CPC // 2


def _gather_one(tab, idx_v, out_hbm, chunk0, bufs, gsem, wsem):
    def issue_gather(j, b):
        pltpu.async_copy(tab.at[idx_v.at[j]], bufs[b], gsem[b])

    def wait_gather(b):
        pltpu.make_async_copy(tab.at[idx_v.at[0]], bufs[b], gsem[b]).wait()

    def wait_write(b):
        pltpu.make_async_copy(bufs[b], out_hbm.at[pl.ds(0, CH)], wsem[b]).wait()

    issue_gather(0, 0)

    @pl.loop(0, PHCH // 2)
    def _(jj):
        for b in range(2):
            j = jj * 2 + b
            ob = 1 - b
            if b == 0:
                @pl.when(jj >= 1)
                def _():
                    wait_write(ob)
            else:
                wait_write(ob)
            wait_gather(b)
            if b == 0:
                issue_gather(j + 1, ob)
            else:
                @pl.when(j + 1 < PHCH)
                def _():
                    issue_gather(j + 1, ob)
            base = (chunk0 + j) * CH
            pltpu.async_copy(bufs[b], out_hbm.at[pl.ds(base, CH)], wsem[b])

    wait_write(1)


def _gather_core(tab, idx_hbm, idx_v, out_hbm, sid, bufs, gsem, wsem):
    for ph in range(2):
        pltpu.sync_copy(idx_hbm.at[pl.ds(sid * CPC + ph * PHCH, PHCH)], idx_v)
        _gather_one(tab, idx_v, out_hbm, sid * CPC + ph * PHCH, bufs, gsem, wsem)


def _gather_body(p_hbm, q_hbm, si_hbm, ri_hbm, gs_hbm, gr_hbm,
                 idx_v, bufs, tab, gsem, wsem):
    c = lax.axis_index("c")
    s = lax.axis_index("s")

    @pl.when(c == 0)
    def _():
        pltpu.sync_copy(p_hbm.at[pl.ds(s * STRIDE, STRIDE)],
                        tab.at[pl.ds(s * STRIDE, STRIDE)])

    @pl.when(c == 1)
    def _():
        pltpu.sync_copy(q_hbm.at[pl.ds(s * STRIDE, STRIDE)],
                        tab.at[pl.ds(s * STRIDE, STRIDE)])

    plsc.subcore_barrier()

    @pl.when(c == 0)
    def _():
        _gather_core(tab, si_hbm, idx_v, gs_hbm, s, bufs, gsem, wsem)

    @pl.when(c == 1)
    def _():
        _gather_core(tab, ri_hbm, idx_v, gr_hbm, s, bufs, gsem, wsem)


def _scatter_body(ne_hbm, ri_hbm, out_hbm, ri_v, vals_v, zer_v, accum, sem):
    c = lax.axis_index("c")
    s = lax.axis_index("s")
    wid = s * NC + c

    @pl.loop(0, CH)
    def _(i):
        @pl.loop(0, LATENT // 16)
        def _(k):
            zer_v[i, pl.ds(k * 16, 16)] = jnp.zeros((16,), _f32)

    @pl.loop(0, NPAD // NS // CH)
    def _(t):
        pltpu.sync_copy(zer_v, accum.at[pl.ds(s * (NPAD // NS) + t * CH, CH)])

    plsc.subcore_barrier()
    pltpu.sync_copy(ri_hbm.at[pl.ds(wid * CPW, CPW)], ri_v)

    @pl.loop(0, CPW)
    def _(j):
        pltpu.sync_copy(ne_hbm.at[pl.ds(wid * RPW + j * CH, CH)], vals_v)
        pltpu.sync_copy(vals_v, accum.at[ri_v.at[j]], add=True)

    plsc.subcore_barrier()

    @pl.loop(0, NPAD // NS // CH)
    def _(t):
        off = s * (NPAD // NS) + t * CH
        pltpu.sync_copy(accum.at[pl.ds(off, CH)], out_hbm.at[c].at[pl.ds(off, CH)])


_gather_call = pl.kernel(
    _gather_body,
    out_type=[jax.ShapeDtypeStruct((EP, LATENT), _f32),
              jax.ShapeDtypeStruct((EP, LATENT), _f32)],
    mesh=_SC_MESH,
    scratch_types=[
        pltpu.VMEM((PHCH, CH), jnp.int32),
        [pltpu.VMEM((CH, LATENT), _f32)] * 2,
        pltpu.VMEM_SHARED((NPAD, LATENT), _f32),
        [pltpu.SemaphoreType.DMA] * 2,
        [pltpu.SemaphoreType.DMA] * 2,
    ],
)

_scatter_call = pl.kernel(
    _scatter_body,
    out_type=jax.ShapeDtypeStruct((NC, NPAD, LATENT), _f32),
    mesh=_SC_MESH,
    scratch_types=[
        pltpu.VMEM((CPW, CH), jnp.int32),
        pltpu.VMEM((CH, LATENT), _f32),
        pltpu.VMEM((CH, LATENT), _f32),
        pltpu.VMEM_SHARED((NPAD, LATENT), _f32),
        pltpu.SemaphoreType.DMA,
    ],
)


def _mlp_weights(p, w1):
    return (w1, p['b1'].reshape(1, -1), p['w2'], p['b2'].reshape(1, -1),
            p['g'].reshape(1, -1), p['b'].reshape(1, -1))


def kernel(node_features, mesh_edge_features, senders, receivers, params):
    p = params
    si = jnp.concatenate(
        [senders.astype(jnp.int32), jnp.zeros((EP - E,), jnp.int32)]
    ).reshape(IDXROWS, CH)
    ri = jnp.concatenate(
        [receivers.astype(jnp.int32), jnp.zeros((EP - E,), jnp.int32)]
    ).reshape(IDXROWS, CH)
    nf = jnp.zeros((NPAD, 16), _f32).at[:N, :NF].set(node_features)

    ne = p['node_enc']
    w1n_enc = jnp.zeros((16, LATENT), _f32).at[:NF].set(ne['w1'])
    nw = _mlp_weights(ne, w1n_enc)
    nl = _tc_call(
        _enc_body, NPAD // NBLK,
        (nf,) + nw,
        [_row_spec(NBLK, 16)] + [_full_spec(w.shape) for w in nw],
        jax.ShapeDtypeStruct((NPAD, LATENT), _f32),
        _row_spec(NBLK, LATENT),
    )
    ee = p['edge_enc']
    ew = _mlp_weights(ee, ee['w1'])
    el0 = _tc_call(
        _enc_body, E // 1000,
        (mesh_edge_features,) + ew,
        [_row_spec(1000, EF)] + [_full_spec(w.shape) for w in ew],
        jax.ShapeDtypeStruct((EP, LATENT), jnp.bfloat16),
        _row_spec(1000, LATENT),
    )

    el_terms = [el0]
    for blk in p['blocks']:
        bw = blk['edge']
        P, Q = _tc_call(
            _proj_body, NPAD // NBLK,
            (nl, bw['w1'][:LATENT], bw['w1'][LATENT:2 * LATENT]),
            [_row_spec(NBLK, LATENT), _full_spec((LATENT, LATENT)),
             _full_spec((LATENT, LATENT))],
            [jax.ShapeDtypeStruct((NPAD, LATENT), _f32)] * 2,
            [_row_spec(NBLK, LATENT)] * 2,
        )
        Gs, Gr = _gather_call(P, Q, si, ri)
        eweights = _mlp_weights(bw, bw['w1'][2 * LATENT:])
        new_e = _tc_call(
            functools.partial(_edge_body, len(el_terms)),
            EP // EBLK,
            (Gs, Gr, *el_terms) + eweights,
            [_row_spec(EBLK, LATENT)] * (2 + len(el_terms))
            + [_full_spec(w.shape) for w in eweights],
            jax.ShapeDtypeStruct((EP, LATENT), _f32),
            _row_spec(EBLK, LATENT),
        )
        el_terms.append(new_e)
        partials = _scatter_call(new_e, ri)
        nb = blk['node']
        nweights = _mlp_weights(nb, nb['w1'][LATENT:])
        nl = _tc_call(
            _node_body, NPAD // NBLK,
            (nl, partials[0], partials[1], nb['w1'][:LATENT]) + nweights,
            [_row_spec(NBLK, LATENT)] * 3 + [_full_spec((LATENT, LATENT))]
            + [_full_spec(w.shape) for w in nweights],
            jax.ShapeDtypeStruct((NPAD, LATENT), _f32),
            _row_spec(NBLK, LATENT),
        )

    out = _tc_call(
        _dec_body, N // 1000,
        (nl, p['c1w'], p['c1b'].reshape(1, -1), p['c2w'], p['c2b'].reshape(1, -1)),
        [_row_spec(1000, LATENT), _full_spec((LATENT, 8)), _full_spec((1, 8)),
         _full_spec((8, TW)), _full_spec((1, TW))],
        jax.ShapeDtypeStruct((N, TW), _f32),
        _row_spec(1000, TW),
    )
    return out


# speedup vs baseline: 3.7805x; 1.1053x over previous
import functools

import jax
import jax.numpy as jnp
from jax import lax
from jax.experimental import pallas as pl
from jax.experimental.pallas import tpu as pltpu
from jax.experimental.pallas import tpu_sc as plsc

LATENT = 128
TW = 5
N = 10000
E = 320000
NF = 11
EF = 4

NPAD = 10240
NC, NS = 2, 16
NW = NC * NS
CH = 128
IDXROWS = 2560
EP = IDXROWS * CH
HALF = IDXROWS // 2
CPS = HALF // NS
CPW = HALF // NW
STRIDE = NPAD // NS

EBLK = 1024
NBLK = 1024

_f32 = jnp.float32
_bf16 = jnp.bfloat16


def _ln(h, g, b):
    mu = jnp.mean(h, axis=-1, keepdims=True)
    var = jnp.mean((h - mu) ** 2, axis=-1, keepdims=True)
    return (h - mu) / jnp.sqrt(var + 1e-5) * g + b


def _enc_body(x_ref, w1_ref, b1_ref, w2_ref, b2_ref, g_ref, b_ref, o_ref):
    h = jnp.maximum(x_ref[...] @ w1_ref[...] + b1_ref[...], 0)
    h = jnp.maximum(h @ w2_ref[...] + b2_ref[...], 0)
    o_ref[...] = _ln(h, g_ref[...], b_ref[...]).astype(o_ref.dtype)


def _proj_body(nl_ref, w1s_ref, w1r_ref, p_ref, q_ref):
    nl = nl_ref[...]
    p_ref[...] = nl @ w1s_ref[...]
    q_ref[...] = nl @ w1r_ref[...]


def _edge_mask(off, y):
    row = (pl.program_id(0) + off) * EBLK + lax.broadcasted_iota(
        jnp.int32, (EBLK, 1), 0)
    return jnp.where(row < E, y, 0.0)


def _edge1_body(off, gs_ref, gr_ref, el_ref, w1e_ref, b1_ref, w2_ref, b2_ref,
                g_ref, b_ref, ne_ref, el1_ref):
    el = el_ref[...].astype(_f32)
    h = jnp.maximum(gs_ref[...] + gr_ref[...] + el @ w1e_ref[...] + b1_ref[...], 0)
    h = jnp.maximum(h @ w2_ref[...] + b2_ref[...], 0)
    ne = _edge_mask(off, _ln(h, g_ref[...], b_ref[...]))
    ne_ref[...] = ne
    el1_ref[...] = (el + ne).astype(_bf16)


def _edge2_body(off, gs_ref, gr_ref, el_ref, w1e_ref, b1_ref, w2_ref, b2_ref,
                g_ref, b_ref, ne_ref):
    el = el_ref[...].astype(_f32)
    h = jnp.maximum(gs_ref[...] + gr_ref[...] + el @ w1e_ref[...] + b1_ref[...], 0)
    h = jnp.maximum(h @ w2_ref[...] + b2_ref[...], 0)
    ne_ref[...] = _edge_mask(off, _ln(h, g_ref[...], b_ref[...]))


def _node_body(nl_ref, a0_ref, a1_ref, a2_ref, a3_ref, w1n_ref, w1a_ref,
               b1_ref, w2_ref, b2_ref, g_ref, b_ref, o_ref):
    nl = nl_ref[...]
    aggr = (a0_ref[...] + a1_ref[...]) + (a2_ref[...] + a3_ref[...])
    h = jnp.maximum(nl @ w1n_ref[...] + aggr @ w1a_ref[...] + b1_ref[...], 0)
    h = jnp.maximum(h @ w2_ref[...] + b2_ref[...], 0)
    o_ref[...] = _ln(h, g_ref[...], b_ref[...]) + nl


def _dec_body(nl_ref, c1w_ref, c1b_ref, c2w_ref, c2b_ref, o_ref):
    h = nl_ref[...] @ c1w_ref[...] + c1b_ref[...]
    h = h * jax.nn.sigmoid(h)
    d = h @ c2w_ref[...] + c2b_ref[...]
    dt = lax.broadcasted_iota(jnp.int32, (1, TW), 1).astype(_f32) + 1.0
    o_ref[...] = d * dt


def _row_spec(blk, width, off=0):
    return pl.BlockSpec((blk, width), lambda i: (i + off, 0))


def _full_spec(shape):
    nd = len(shape)
    return pl.BlockSpec(shape, lambda i: (0,) * nd)


def _tc_call(body, grid, in_arrays, in_specs, out_shape, out_specs):
    return pl.pallas_call(
        body,
        grid=(grid,),
        in_specs=in_specs,
        out_specs=out_specs,
        out_shape=out_shape,
    )(*in_arrays)


_SC_MESH = plsc.VectorSubcoreMesh(core_axis_name="c", subcore_axis_name="s")


def _gather_one(tab, idx_v, out_hbm, chunk0, bufs, gsem, wsem):
    def issue_gather(j, b):
        pltpu.async_copy(tab.at[idx_v.at[j]], bufs[b], gsem[b])

    def wait_gather(b):
        pltpu.make_async_copy(tab.at[idx_v.at[0]], bufs[b], gsem[b]).wait()

    def wait_write(b):
        pltpu.make_async_copy(bufs[b], out_hbm.at[pl.ds(0, CH)], wsem[b]).wait()

    issue_gather(0, 0)

    @pl.loop(0, CPS // 2)
    def _(jj):
        for b in range(2):
            j = jj * 2 + b
            ob = 1 - b
            if b == 0:
                @pl.when(jj >= 1)
                def _():
                    wait_write(ob)
            else:
                wait_write(ob)
            wait_gather(b)
            if b == 0:
                issue_gather(j + 1, ob)
            else:
                @pl.when(j + 1 < CPS)
                def _():
                    issue_gather(j + 1, ob)
            base = (chunk0 + j) * CH
            pltpu.async_copy(bufs[b], out_hbm.at[pl.ds(base, CH)], wsem[b])

    wait_write(1)


def _gather_body(half, p_hbm, q_hbm, si_hbm, ri_hbm, gs_hbm, gr_hbm,
                 idx_v, bufs, tab, gsem, wsem):
    c = lax.axis_index("c")
    s = lax.axis_index("s")

    @pl.when(c == 0)
    def _():
        pltpu.sync_copy(p_hbm.at[pl.ds(s * STRIDE, STRIDE)],
                        tab.at[pl.ds(s * STRIDE, STRIDE)])

    @pl.when(c == 1)
    def _():
        pltpu.sync_copy(q_hbm.at[pl.ds(s * STRIDE, STRIDE)],
                        tab.at[pl.ds(s * STRIDE, STRIDE)])

    plsc.subcore_barrier()
    chunk0 = half * HALF + s * CPS

    @pl.when(c == 0)
    def _():
        pltpu.sync_copy(si_hbm.at[pl.ds(chunk0, CPS)], idx_v)
        _gather_one(tab, idx_v, gs_hbm, chunk0, bufs, gsem, wsem)

    @pl.when(c == 1)
    def _():
        pltpu.sync_copy(ri_hbm.at[pl.ds(chunk0, CPS)], idx_v)
        _gather_one(tab, idx_v, gr_hbm, chunk0, bufs, gsem, wsem)


def _scatter_body(half, ne_hbm, ri_hbm, out_hbm, ri_v, vals_v, zer_v, accum,
                  sem):
    c = lax.axis_index("c")
    s = lax.axis_index("s")
    wid = s * NC + c

    @pl.loop(0, CH)
    def _(i):
        @pl.loop(0, LATENT // 16)
        def _(k):
            zer_v[i, pl.ds(k * 16, 16)] = jnp.zeros((16,), _f32)

    @pl.loop(0, STRIDE // CH)
    def _(t):
        pltpu.sync_copy(zer_v, accum.at[pl.ds(s * STRIDE + t * CH, CH)])

    plsc.subcore_barrier()
    chunk0 = half * HALF + wid * CPW
    pltpu.sync_copy(ri_hbm.at[pl.ds(chunk0, CPW)], ri_v)

    @pl.loop(0, CPW)
    def _(j):
        pltpu.sync_copy(ne_hbm.at[pl.ds((chunk0 + j) * CH, CH)], vals_v)
        pltpu.sync_copy(vals_v, accum.at[ri_v.at[j]], add=True)

    plsc.subcore_barrier()

    @pl.loop(0, STRIDE // CH)
    def _(t):
        off = s * STRIDE + t * CH
        pltpu.sync_copy(accum.at[pl.ds(off, CH)], out_hbm.at[c].at[pl.ds(off, CH)])


def _make_gather(half):
    return pl.kernel(
        functools.partial(_gather_body, half),
        out_type=[jax.ShapeDtypeStruct((EP, LATENT), _f32),
                  jax.ShapeDtypeStruct((EP, LATENT), _f32)],
        mesh=_SC_MESH,
        scratch_types=[
            pltpu.VMEM((CPS, CH), jnp.int32),
            [pltpu.VMEM((CH, LATENT), _f32)] * 2,
            pltpu.VMEM_SHARED((NPAD, LATENT), _f32),
            [pltpu.SemaphoreType.DMA] * 2,
            [pltpu.SemaphoreType.DMA] * 2,
        ],
    )


def _make_scatter(half):
    return pl.kernel(
        functools.partial(_scatter_body, half),
        out_type=jax.ShapeDtypeStruct((NC, NPAD, LATENT), _f32),
        mesh=_SC_MESH,
        scratch_types=[
            pltpu.VMEM((CPW, CH), jnp.int32),
            pltpu.VMEM((CH, LATENT), _f32),
            pltpu.VMEM((CH, LATENT), _f32),
            pltpu.VMEM_SHARED((NPAD, LATENT), _f32),
            pltpu.SemaphoreType.DMA,
        ],
    )


_gather_calls = [_make_gather(0), _make_gather(1)]
_scatter_calls = [_make_scatter(0), _make_scatter(1)]


def _mlp_weights(p, w1):
    return (w1, p['b1'].reshape(1, -1), p['w2'], p['b2'].reshape(1, -1),
            p['g'].reshape(1, -1), p['b'].reshape(1, -1))


def _edge_half(body, half, n_out, in_arrays, weights, out_shapes):
    off = half * (HALF * CH // EBLK)
    n_rows = len(in_arrays)
    return pl.pallas_call(
        functools.partial(body, off),
        grid=(HALF * CH // EBLK,),
        in_specs=[_row_spec(EBLK, LATENT, off)] * n_rows
        + [_full_spec(w.shape) for w in weights],
        out_specs=[_row_spec(EBLK, LATENT, off)] * n_out,
        out_shape=out_shapes,
    )(*in_arrays, *weights)


def kernel(node_features, mesh_edge_features, senders, receivers, params):
    p = params
    si = jnp.concatenate(
        [senders.astype(jnp.int32), jnp.zeros((EP - E,), jnp.int32)]
    ).reshape(IDXROWS, CH)
    ri = jnp.concatenate(
        [receivers.astype(jnp.int32), jnp.zeros((EP - E,), jnp.int32)]
    ).reshape(IDXROWS, CH)
    nf = jnp.zeros((NPAD, 16), _f32).at[:N, :NF].set(node_features)

    ne = p['node_enc']
    w1n_enc = jnp.zeros((16, LATENT), _f32).at[:NF].set(ne['w1'])
    nw = _mlp_weights(ne, w1n_enc)
    nl = _tc_call(
        _enc_body, NPAD // NBLK,
        (nf,) + nw,
        [_row_spec(NBLK, 16)] + [_full_spec(w.shape) for w in nw],
        jax.ShapeDtypeStruct((NPAD, LATENT), _f32),
        _row_spec(NBLK, LATENT),
    )
    ee = p['edge_enc']
    ew = _mlp_weights(ee, ee['w1'])
    el0 = _tc_call(
        _enc_body, E // 1000,
        (mesh_edge_features,) + ew,
        [_row_spec(1000, EF)] + [_full_spec(w.shape) for w in ew],
        jax.ShapeDtypeStruct((EP, LATENT), _bf16),
        _row_spec(1000, LATENT),
    )

    el_halves = [el0, el0]
    for bi, blk in enumerate(p['blocks']):
        bw = blk['edge']
        P, Q = _tc_call(
            _proj_body, NPAD // NBLK,
            (nl, bw['w1'][:LATENT], bw['w1'][LATENT:2 * LATENT]),
            [_row_spec(NBLK, LATENT), _full_spec((LATENT, LATENT)),
             _full_spec((LATENT, LATENT))],
            [jax.ShapeDtypeStruct((NPAD, LATENT), _f32)] * 2,
            [_row_spec(NBLK, LATENT)] * 2,
        )
        eweights = _mlp_weights(bw, bw['w1'][2 * LATENT:])
        gs0, gr0 = _gather_calls[0](P, Q, si, ri)
        gs1, gr1 = _gather_calls[1](P, Q, si, ri)
        el1_halves, partials = [], []
        for h, (gs, gr) in enumerate(((gs0, gr0), (gs1, gr1))):
            if bi == 0:
                ne_h, el1_h = _edge_half(
                    _edge1_body, h, 2, (gs, gr, el_halves[h]), eweights,
                    [jax.ShapeDtypeStruct((EP, LATENT), _f32),
                     jax.ShapeDtypeStruct((EP, LATENT), _bf16)],
                )
                el1_halves.append(el1_h)
            else:
                (ne_h,) = _edge_half(
                    _edge2_body, h, 1, (gs, gr, el_halves[h]), eweights,
                    [jax.ShapeDtypeStruct((EP, LATENT), _f32)],
                )
            partials.append(_scatter_calls[h](ne_h, ri))
        if bi == 0:
            el_halves = el1_halves
        nb = blk['node']
        nweights = _mlp_weights(nb, nb['w1'][LATENT:])
        nl = _tc_call(
            _node_body, NPAD // NBLK,
            (nl, partials[0][0], partials[0][1], partials[1][0],
             partials[1][1], nb['w1'][:LATENT]) + nweights,
            [_row_spec(NBLK, LATENT)] * 5 + [_full_spec((LATENT, LATENT))]
            + [_full_spec(w.shape) for w in nweights],
            jax.ShapeDtypeStruct((NPAD, LATENT), _f32),
            _row_spec(NBLK, LATENT),
        )

    out = _tc_call(
        _dec_body, N // 1000,
        (nl, p['c1w'], p['c1b'].reshape(1, -1), p['c2w'], p['c2b'].reshape(1, -1)),
        [_row_spec(1000, LATENT), _full_spec((LATENT, 8)), _full_spec((1, 8)),
         _full_spec((8, TW)), _full_spec((1, TW))],
        jax.ShapeDtypeStruct((N, TW), _f32),
        _row_spec(1000, TW),
    )
    return out


# speedup vs baseline: 3.9273x; 1.0388x over previous
import functools

import jax
import jax.numpy as jnp
from jax import lax
from jax.experimental import pallas as pl
from jax.experimental.pallas import tpu as pltpu
from jax.experimental.pallas import tpu_sc as plsc

LATENT = 128
TW = 5
N = 10000
E = 320000
NF = 11
EF = 4

NPAD = 10240
NC, NS = 2, 16
NW = NC * NS
CH = 128
IDXROWS = 2560
EP = IDXROWS * CH
HALF = IDXROWS // 2
CPS = HALF // NS
CPW = HALF // NW
STRIDE = NPAD // NS

EBLK = 1024
NBLK = 1024

_f32 = jnp.float32
_bf16 = jnp.bfloat16


def _ln(h, g, b):
    mu = jnp.mean(h, axis=-1, keepdims=True)
    var = jnp.mean((h - mu) ** 2, axis=-1, keepdims=True)
    return (h - mu) / jnp.sqrt(var + 1e-5) * g + b


def _enc_body(x_ref, w1_ref, b1_ref, w2_ref, b2_ref, g_ref, b_ref, o_ref):
    h = jnp.maximum(x_ref[...] @ w1_ref[...] + b1_ref[...], 0)
    h = jnp.maximum(h @ w2_ref[...] + b2_ref[...], 0)
    o_ref[...] = _ln(h, g_ref[...], b_ref[...]).astype(o_ref.dtype)


ENC_GRID = E // 1000


def _edge_enc_body(ef_hbm, w1_ref, b1_ref, w2_ref, b2_ref, g_ref, b_ref,
                   o_ref, xb0, xb1, sem0, sem1):
    i = pl.program_id(0)

    def blk(j):
        return ef_hbm.at[pl.ds(j * 1000, 1000), :]

    @pl.when(i == 0)
    def _():
        pltpu.async_copy(blk(0), xb0, sem0)
        pltpu.async_copy(blk(1), xb1, sem1)

    even = i % 2 == 0

    @pl.when(even)
    def _():
        pltpu.make_async_copy(blk(0), xb0, sem0).wait()

    @pl.when(jnp.logical_not(even))
    def _():
        pltpu.make_async_copy(blk(0), xb1, sem1).wait()

    x = jnp.where(even, xb0[...], xb1[...])

    @pl.when(even & (i + 2 < ENC_GRID))
    def _():
        pltpu.async_copy(blk(i + 2), xb0, sem0)

    @pl.when(jnp.logical_not(even) & (i + 2 < ENC_GRID))
    def _():
        pltpu.async_copy(blk(i + 2), xb1, sem1)

    h = jnp.maximum(x @ w1_ref[...] + b1_ref[...], 0)
    h = jnp.maximum(h @ w2_ref[...] + b2_ref[...], 0)
    o_ref[...] = _ln(h, g_ref[...], b_ref[...]).astype(o_ref.dtype)


def _proj_body(nl_ref, w1s_ref, w1r_ref, p_ref, q_ref):
    nl = nl_ref[...]
    p_ref[...] = nl @ w1s_ref[...]
    q_ref[...] = nl @ w1r_ref[...]


def _edge_mask(off, y):
    row = (pl.program_id(0) + off) * EBLK + lax.broadcasted_iota(
        jnp.int32, (EBLK, 1), 0)
    return jnp.where(row < E, y, 0.0)


def _edge1_body(off, gs_ref, gr_ref, el_ref, w1e_ref, b1_ref, w2_ref, b2_ref,
                g_ref, b_ref, ne_ref, el1_ref):
    el = el_ref[...].astype(_f32)
    h = jnp.maximum(gs_ref[...] + gr_ref[...] + el @ w1e_ref[...] + b1_ref[...], 0)
    h = jnp.maximum(h @ w2_ref[...] + b2_ref[...], 0)
    ne = _edge_mask(off, _ln(h, g_ref[...], b_ref[...]))
    ne_ref[...] = ne
    el1_ref[...] = (el + ne).astype(_bf16)


def _edge2_body(off, gs_ref, gr_ref, el_ref, w1e_ref, b1_ref, w2_ref, b2_ref,
                g_ref, b_ref, ne_ref):
    el = el_ref[...].astype(_f32)
    h = jnp.maximum(gs_ref[...] + gr_ref[...] + el @ w1e_ref[...] + b1_ref[...], 0)
    h = jnp.maximum(h @ w2_ref[...] + b2_ref[...], 0)
    ne_ref[...] = _edge_mask(off, _ln(h, g_ref[...], b_ref[...]))


def _node_body(nl_ref, a0_ref, a1_ref, a2_ref, a3_ref, w1n_ref, w1a_ref,
               b1_ref, w2_ref, b2_ref, g_ref, b_ref, o_ref):
    nl = nl_ref[...]
    aggr = (a0_ref[...] + a1_ref[...]) + (a2_ref[...] + a3_ref[...])
    h = jnp.maximum(nl @ w1n_ref[...] + aggr @ w1a_ref[...] + b1_ref[...], 0)
    h = jnp.maximum(h @ w2_ref[...] + b2_ref[...], 0)
    o_ref[...] = _ln(h, g_ref[...], b_ref[...]) + nl


def _dec_body(nl_ref, c1w_ref, c1b_ref, c2w_ref, c2b_ref, o_ref):
    h = nl_ref[...] @ c1w_ref[...] + c1b_ref[...]
    h = h * jax.nn.sigmoid(h)
    d = h @ c2w_ref[...] + c2b_ref[...]
    dt = lax.broadcasted_iota(jnp.int32, (1, TW), 1).astype(_f32) + 1.0
    o_ref[...] = d * dt


def _row_spec(blk, width, off=0):
    return pl.BlockSpec((blk, width), lambda i: (i + off, 0))


def _full_spec(shape):
    nd = len(shape)
    return pl.BlockSpec(shape, lambda i: (0,) * nd)


def _tc_call(body, grid, in_arrays, in_specs, out_shape, out_specs):
    return pl.pallas_call(
        body,
        grid=(grid,),
        in_specs=in_specs,
        out_specs=out_specs,
        out_shape=out_shape,
    )(*in_arrays)


_SC_MESH = plsc.VectorSubcoreMesh(core_axis_name="c", subcore_axis_name="s")


def _gather_one(tab, idx_v, out_hbm, chunk0, bufs, gsem, wsem):
    def issue_gather(j, b):
        pltpu.async_copy(tab.at[idx_v.at[j]], bufs[b], gsem[b])

    def wait_gather(b):
        pltpu.make_async_copy(tab.at[idx_v.at[0]], bufs[b], gsem[b]).wait()

    def wait_write(b):
        pltpu.make_async_copy(bufs[b], out_hbm.at[pl.ds(0, CH)], wsem[b]).wait()

    issue_gather(0, 0)

    @pl.loop(0, CPS // 2)
    def _(jj):
        for b in range(2):
            j = jj * 2 + b
            ob = 1 - b
            if b == 0:
                @pl.when(jj >= 1)
                def _():
                    wait_write(ob)
            else:
                wait_write(ob)
            wait_gather(b)
            if b == 0:
                issue_gather(j + 1, ob)
            else:
                @pl.when(j + 1 < CPS)
                def _():
                    issue_gather(j + 1, ob)
            base = (chunk0 + j) * CH
            pltpu.async_copy(bufs[b], out_hbm.at[pl.ds(base, CH)], wsem[b])

    wait_write(1)


def _gather_body(half, p_hbm, q_hbm, si_hbm, ri_hbm, gs_hbm, gr_hbm,
                 idx_v, bufs, tab, gsem, wsem):
    c = lax.axis_index("c")
    s = lax.axis_index("s")

    @pl.when(c == 0)
    def _():
        pltpu.sync_copy(p_hbm.at[pl.ds(s * STRIDE, STRIDE)],
                        tab.at[pl.ds(s * STRIDE, STRIDE)])

    @pl.when(c == 1)
    def _():
        pltpu.sync_copy(q_hbm.at[pl.ds(s * STRIDE, STRIDE)],
                        tab.at[pl.ds(s * STRIDE, STRIDE)])

    plsc.subcore_barrier()
    chunk0 = half * HALF + s * CPS

    @pl.when(c == 0)
    def _():
        pltpu.sync_copy(si_hbm.at[pl.ds(chunk0, CPS)], idx_v)
        _gather_one(tab, idx_v, gs_hbm, chunk0, bufs, gsem, wsem)

    @pl.when(c == 1)
    def _():
        pltpu.sync_copy(ri_hbm.at[pl.ds(chunk0, CPS)], idx_v)
        _gather_one(tab, idx_v, gr_hbm, chunk0, bufs, gsem, wsem)


def _scatter_body(half, ne_hbm, ri_hbm, out_hbm, ri_v, vals_v, zer_v, accum,
                  sem):
    c = lax.axis_index("c")
    s = lax.axis_index("s")
    wid = s * NC + c

    @pl.loop(0, CH)
    def _(i):
        @pl.loop(0, LATENT // 16)
        def _(k):
            zer_v[i, pl.ds(k * 16, 16)] = jnp.zeros((16,), _f32)

    @pl.loop(0, STRIDE // CH)
    def _(t):
        pltpu.sync_copy(zer_v, accum.at[pl.ds(s * STRIDE + t * CH, CH)])

    plsc.subcore_barrier()
    chunk0 = half * HALF + wid * CPW
    pltpu.sync_copy(ri_hbm.at[pl.ds(chunk0, CPW)], ri_v)

    @pl.loop(0, CPW)
    def _(j):
        pltpu.sync_copy(ne_hbm.at[pl.ds((chunk0 + j) * CH, CH)], vals_v)
        pltpu.sync_copy(vals_v, accum.at[ri_v.at[j]], add=True)

    plsc.subcore_barrier()

    @pl.loop(0, STRIDE // CH)
    def _(t):
        off = s * STRIDE + t * CH
        pltpu.sync_copy(accum.at[pl.ds(off, CH)], out_hbm.at[c].at[pl.ds(off, CH)])


def _make_gather(half):
    return pl.kernel(
        functools.partial(_gather_body, half),
        out_type=[jax.ShapeDtypeStruct((EP, LATENT), _f32),
                  jax.ShapeDtypeStruct((EP, LATENT), _f32)],
        mesh=_SC_MESH,
        scratch_types=[
            pltpu.VMEM((CPS, CH), jnp.int32),
            [pltpu.VMEM((CH, LATENT), _f32)] * 2,
            pltpu.VMEM_SHARED((NPAD, LATENT), _f32),
            [pltpu.SemaphoreType.DMA] * 2,
            [pltpu.SemaphoreType.DMA] * 2,
        ],
    )


def _make_scatter(half):
    return pl.kernel(
        functools.partial(_scatter_body, half),
        out_type=jax.ShapeDtypeStruct((NC, NPAD, LATENT), _f32),
        mesh=_SC_MESH,
        scratch_types=[
            pltpu.VMEM((CPW, CH), jnp.int32),
            pltpu.VMEM((CH, LATENT), _f32),
            pltpu.VMEM((CH, LATENT), _f32),
            pltpu.VMEM_SHARED((NPAD, LATENT), _f32),
            pltpu.SemaphoreType.DMA,
        ],
    )


_gather_calls = [_make_gather(0), _make_gather(1)]
_scatter_calls = [_make_scatter(0), _make_scatter(1)]


def _mlp_weights(p, w1):
    return (w1, p['b1'].reshape(1, -1), p['w2'], p['b2'].reshape(1, -1),
            p['g'].reshape(1, -1), p['b'].reshape(1, -1))


def _edge_half(body, half, n_out, in_arrays, weights, out_shapes):
    off = half * (HALF * CH // EBLK)
    n_rows = len(in_arrays)
    return pl.pallas_call(
        functools.partial(body, off),
        grid=(HALF * CH // EBLK,),
        in_specs=[_row_spec(EBLK, LATENT, off)] * n_rows
        + [_full_spec(w.shape) for w in weights],
        out_specs=[_row_spec(EBLK, LATENT, off)] * n_out,
        out_shape=out_shapes,
    )(*in_arrays, *weights)


def kernel(node_features, mesh_edge_features, senders, receivers, params):
    p = params
    si = jnp.concatenate(
        [senders.astype(jnp.int32), jnp.zeros((EP - E,), jnp.int32)]
    ).reshape(IDXROWS, CH)
    ri = jnp.concatenate(
        [receivers.astype(jnp.int32), jnp.zeros((EP - E,), jnp.int32)]
    ).reshape(IDXROWS, CH)
    nf = jnp.zeros((NPAD, 16), _f32).at[:N, :NF].set(node_features)

    ne = p['node_enc']
    w1n_enc = jnp.zeros((16, LATENT), _f32).at[:NF].set(ne['w1'])
    nw = _mlp_weights(ne, w1n_enc)
    nl = _tc_call(
        _enc_body, NPAD // NBLK,
        (nf,) + nw,
        [_row_spec(NBLK, 16)] + [_full_spec(w.shape) for w in nw],
        jax.ShapeDtypeStruct((NPAD, LATENT), _f32),
        _row_spec(NBLK, LATENT),
    )
    ee = p['edge_enc']
    ew = _mlp_weights(ee, ee['w1'])
    el0 = pl.pallas_call(
        _edge_enc_body,
        grid=(ENC_GRID,),
        in_specs=[pl.BlockSpec(memory_space=pl.ANY)]
        + [_full_spec(w.shape) for w in ew],
        out_specs=_row_spec(1000, LATENT),
        out_shape=jax.ShapeDtypeStruct((EP, LATENT), _bf16),
        scratch_shapes=[
            pltpu.VMEM((1000, EF), _f32),
            pltpu.VMEM((1000, EF), _f32),
            pltpu.SemaphoreType.DMA,
            pltpu.SemaphoreType.DMA,
        ],
    )(mesh_edge_features, *ew)

    el_halves = [el0, el0]
    for bi, blk in enumerate(p['blocks']):
        bw = blk['edge']
        P, Q = _tc_call(
            _proj_body, NPAD // NBLK,
            (nl, bw['w1'][:LATENT], bw['w1'][LATENT:2 * LATENT]),
            [_row_spec(NBLK, LATENT), _full_spec((LATENT, LATENT)),
             _full_spec((LATENT, LATENT))],
            [jax.ShapeDtypeStruct((NPAD, LATENT), _f32)] * 2,
            [_row_spec(NBLK, LATENT)] * 2,
        )
        eweights = _mlp_weights(bw, bw['w1'][2 * LATENT:])
        gs0, gr0 = _gather_calls[0](P, Q, si, ri)
        gs1, gr1 = _gather_calls[1](P, Q, si, ri)
        el1_halves, partials = [], []
        for h, (gs, gr) in enumerate(((gs0, gr0), (gs1, gr1))):
            if bi == 0:
                ne_h, el1_h = _edge_half(
                    _edge1_body, h, 2, (gs, gr, el_halves[h]), eweights,
                    [jax.ShapeDtypeStruct((EP, LATENT), _f32),
                     jax.ShapeDtypeStruct((EP, LATENT), _bf16)],
                )
                el1_halves.append(el1_h)
            else:
                (ne_h,) = _edge_half(
                    _edge2_body, h, 1, (gs, gr, el_halves[h]), eweights,
                    [jax.ShapeDtypeStruct((EP, LATENT), _f32)],
                )
            partials.append(_scatter_calls[h](ne_h, ri))
        if bi == 0:
            el_halves = el1_halves
        nb = blk['node']
        nweights = _mlp_weights(nb, nb['w1'][LATENT:])
        nl = _tc_call(
            _node_body, NPAD // NBLK,
            (nl, partials[0][0], partials[0][1], partials[1][0],
             partials[1][1], nb['w1'][:LATENT]) + nweights,
            [_row_spec(NBLK, LATENT)] * 5 + [_full_spec((LATENT, LATENT))]
            + [_full_spec(w.shape) for w in nweights],
            jax.ShapeDtypeStruct((NPAD, LATENT), _f32),
            _row_spec(NBLK, LATENT),
        )

    out = _tc_call(
        _dec_body, N // 1000,
        (nl, p['c1w'], p['c1b'].reshape(1, -1), p['c2w'], p['c2b'].reshape(1, -1)),
        [_row_spec(1000, LATENT), _full_spec((LATENT, 8)), _full_spec((1, 8)),
         _full_spec((8, TW)), _full_spec((1, TW))],
        jax.ShapeDtypeStruct((N, TW), _f32),
        _row_spec(1000, TW),
    )
    return out
